# baseline (speedup 1.0000x reference)
import jax
import jax.numpy as jnp
from jax import lax
from jax.experimental import pallas as pl
from jax.experimental.pallas import tpu as pltpu
from jax.experimental.pallas import tpu_sc as plsc

N_NODES = 10000
N_EDGES = 160000
D_FEAT = 128
D_EDGE = 4

NUM_CORES = 2
NUM_SUBCORES = 16
EB = 64
EDGES_PER_TILE = 10240
E_PAD = NUM_SUBCORES * EDGES_PER_TILE
NB = EDGES_PER_TILE // EB
WROWS = EB * 32 // 128
IROWS = EB * 16 // 128
SROWS = E_PAD * 32 // 128
STRIPE = 632
STRIPE_LAST = N_NODES - 15 * STRIPE


def _sc_body(in1_hbm, wq0, wq1, wq2, wq3, in2t_hbm, sd_hbm,
             zrows_hbm, out_hbm, stash0_hbm, stash1_hbm, accum,
             sidx, didx, wt, i2, rows, msg, stash,
             in_sems, g_sems, s_sems, st_sem, si_sems):
  cid = lax.axis_index("c")
  sid = lax.axis_index("s")
  ebase0 = sid * EDGES_PER_TILE
  rbase = pl.multiple_of(sid * STRIPE, 8)

  def stripe_copy(src_fn, dst_fn):
    @pl.when(sid < NUM_SUBCORES - 1)
    def _():
      pltpu.sync_copy(src_fn(rbase, STRIPE), dst_fn(rbase, STRIPE))

    @pl.when(sid == NUM_SUBCORES - 1)
    def _():
      base = (NUM_SUBCORES - 1) * STRIPE
      pltpu.sync_copy(src_fn(base, STRIPE_LAST), dst_fn(base, STRIPE_LAST))

  def eoff(b):
    return pl.multiple_of(ebase0 + b * EB, 8)

  def ioff(b):
    return pl.multiple_of((ebase0 + b * EB) // 8, 8)

  def woff(b):
    return pl.multiple_of((ebase0 + b * EB) // 4, 8)

  def issue_scatter(half, dslot):
    pltpu.async_copy(msg.at[half], accum.at[didx.at[dslot]],
                     s_sems.at[half], add=True)

  def wait_scatter(half, dslot):
    pltpu.make_async_copy(msg.at[half], accum.at[didx.at[dslot]],
                          s_sems.at[half]).wait()

  def epilogue(plane):
    wait_scatter(0, 2)
    wait_scatter(1, 3)
    plsc.subcore_barrier()
    stripe_copy(lambda base, n: accum.at[pl.ds(base, n), :],
                lambda base, n: out_hbm.at[plane, pl.ds(base, n), :])
    plsc.subcore_barrier()

  def zero_accum():
    stripe_copy(lambda base, n: zrows_hbm.at[pl.ds(0, n), :],
                lambda base, n: accum.at[pl.ds(base, n), :])
    plsc.subcore_barrier()

  def run_pass1(wtblA, wtblB, c0, stash_hbm, plane):
    zero_accum()

    def stage1_descs(b, half, dslot):
      sem = in_sems.at[half]
      return [
          pltpu.make_async_copy(sd_hbm.at[pl.ds(E_PAD + eoff(b), EB)],
                                didx.at[dslot], sem),
          pltpu.make_async_copy(in2t_hbm.at[pl.ds(ioff(b), IROWS), :],
                                i2.at[half], sem),
          pltpu.make_async_copy(wtblA.at[pl.ds(woff(b), WROWS), :],
                                wt.at[half, pl.ds(0, WROWS), :], sem),
          pltpu.make_async_copy(wtblB.at[pl.ds(woff(b), WROWS), :],
                                wt.at[half, pl.ds(WROWS, WROWS), :], sem),
      ]

    def stage1(b, half, dslot):
      for d in stage1_descs(b, half, dslot):
        d.start()

    def wait_stage1(b, half, dslot):
      for d in stage1_descs(b, half, dslot):
        d.wait()

    def sidx_desc(b, sslot):
      return pltpu.make_async_copy(sd_hbm.at[pl.ds(eoff(b), EB)],
                                   sidx.at[sslot], si_sems.at[sslot % 2])

    def issue_gather(sslot, half):
      pltpu.async_copy(in1_hbm.at[sidx.at[sslot]], rows.at[half],
                       g_sems.at[half])

    def wait_gather(sslot, half):
      pltpu.make_async_copy(in1_hbm.at[sidx.at[sslot]], rows.at[half],
                            g_sems.at[half]).wait()

    def stash_desc(b):
      return pltpu.make_async_copy(
          stash, stash_hbm.at[pl.ds(woff(b), WROWS), :], st_sem)

    def compute(half):
      @plsc.parallel_loop(0, EB // 8, 1, unroll=2)
      def _(g):
        for t in range(8):
          e = 8 * g + t
          wr = 2 * g + t // 4
          ca = 32 * (t % 4)
          wA0 = rows[half, e, c0:c0 + 16] * wt[half, wr, ca:ca + 16]
          wA1 = (rows[half, e, c0 + 16:c0 + 32]
                 * wt[half, wr, ca + 16:ca + 32])
          wB0 = (rows[half, e, c0 + 32:c0 + 48]
                 * wt[half, WROWS + wr, ca:ca + 16])
          wB1 = (rows[half, e, c0 + 48:c0 + 64]
                 * wt[half, WROWS + wr, ca + 16:ca + 32])
          stash[wr, ca:ca + 16] = wB0
          stash[wr, ca + 16:ca + 32] = wB1
          tv = i2[half, g, 16 * t:16 * t + 16]
          for j in range(4):
            s = tv[j]
            msg[half, e, 32 * j:32 * j + 16] = wA0 * s
            msg[half, e, 32 * j + 16:32 * j + 32] = wA1 * s

    sidx_desc(0, 0).start()
    sidx_desc(1, 1).start()
    stage1(0, 0, 0)
    wait_stage1(0, 0, 0)
    sidx_desc(0, 0).wait()
    issue_gather(0, 0)
    sidx_desc(1, 1).wait()
    issue_gather(1, 1)

    def body(i, carry):
      for h in range(4):
        b = 4 * i + h
        half = h % 2

        if h < 2:
          @pl.when(i > 0)
          def _():
            wait_scatter(half, (h + 2) % 4)
        else:
          wait_scatter(half, (h + 2) % 4)

        wait_gather(h, half)

        if h == 3:
          @pl.when(i < NB // 4 - 1)
          def _():
            stage1(b + 1, 1 - half, (h + 1) % 4)
        else:
          stage1(b + 1, 1 - half, (h + 1) % 4)

        if h >= 2:
          @pl.when(i < NB // 4 - 1)
          def _():
            sidx_desc(b + 2, (h + 2) % 4).start()
        else:
          sidx_desc(b + 2, (h + 2) % 4).start()

        if h == 0:
          @pl.when(i > 0)
          def _():
            stash_desc(b - 1).wait()
        else:
          stash_desc(b - 1).wait()

        compute(half)
        stash_desc(b).start()
        issue_scatter(half, h)

        if h == 3:
          @pl.when(i < NB // 4 - 1)
          def _():
            wait_stage1(b + 1, 1 - half, (h + 1) % 4)
        else:
          wait_stage1(b + 1, 1 - half, (h + 1) % 4)

        if h >= 2:
          @pl.when(i < NB // 4 - 1)
          def _():
            sidx_desc(b + 2, (h + 2) % 4).wait()
            issue_gather((h + 2) % 4, half)
        else:
          sidx_desc(b + 2, (h + 2) % 4).wait()
          issue_gather((h + 2) % 4, half)
      return carry

    lax.fori_loop(0, NB // 4, body, 0)
    stash_desc(NB - 1).wait()
    epilogue(plane)

  def run_pass2(stash_hbm, plane):
    zero_accum()

    def stage1_descs(b, half, dslot):
      sem = in_sems.at[half]
      return [
          pltpu.make_async_copy(sd_hbm.at[pl.ds(E_PAD + eoff(b), EB)],
                                didx.at[dslot], sem),
          pltpu.make_async_copy(in2t_hbm.at[pl.ds(ioff(b), IROWS), :],
                                i2.at[half], sem),
          pltpu.make_async_copy(stash_hbm.at[pl.ds(woff(b), WROWS), :],
                                wt.at[half, pl.ds(0, WROWS), :], sem),
      ]

    def stage1(b, half, dslot):
      for d in stage1_descs(b, half, dslot):
        d.start()

    def wait_stage1(b, half, dslot):
      for d in stage1_descs(b, half, dslot):
        d.wait()

    def compute(half):
      @plsc.parallel_loop(0, EB // 8, 1, unroll=2)
      def _(g):
        for t in range(8):
          e = 8 * g + t
          wr = 2 * g + t // 4
          ca = 32 * (t % 4)
          w0 = wt[half, wr, ca:ca + 16]
          w1 = wt[half, wr, ca + 16:ca + 32]
          tv = i2[half, g, 16 * t:16 * t + 16]
          for j in range(4):
            s = tv[j]
            msg[half, e, 32 * j:32 * j + 16] = w0 * s
            msg[half, e, 32 * j + 16:32 * j + 32] = w1 * s

    stage1(0, 0, 0)
    wait_stage1(0, 0, 0)

    def body(i, carry):
      for h in range(4):
        b = 4 * i + h
        half = h % 2

        if h < 2:
          @pl.when(i > 0)
          def _():
            wait_scatter(half, (h + 2) % 4)
        else:
          wait_scatter(half, (h + 2) % 4)

        if h == 3:
          @pl.when(i < NB // 4 - 1)
          def _():
            stage1(b + 1, 1 - half, (h + 1) % 4)
        else:
          stage1(b + 1, 1 - half, (h + 1) % 4)

        compute(half)
        issue_scatter(half, h)

        if h == 3:
          @pl.when(i < NB // 4 - 1)
          def _():
            wait_stage1(b + 1, 1 - half, (h + 1) % 4)
        else:
          wait_stage1(b + 1, 1 - half, (h + 1) % 4)
      return carry

    lax.fori_loop(0, NB // 4, body, 0)
    epilogue(plane)

  @pl.when(cid == 0)
  def _():
    run_pass1(wq0, wq1, 0, stash0_hbm, 0)
    run_pass2(stash0_hbm, 1)

  @pl.when(cid == 1)
  def _():
    run_pass1(wq2, wq3, 64, stash1_hbm, 2)
    run_pass2(stash1_hbm, 3)


@jax.jit
def _fused_uvu(in1, in2, weight, src, dst):
  pad = E_PAD - N_EDGES
  zpad = jnp.zeros((pad,), jnp.int32)
  sd = jnp.concatenate([src, zpad, dst, zpad])
  wp = jnp.concatenate([weight, jnp.zeros((pad, D_FEAT), jnp.float32)])
  wq = jnp.transpose(jnp.reshape(wp, (E_PAD, 4, 32)), (1, 0, 2))
  wqr = jnp.reshape(wq, (4, SROWS, 128))
  in2p = jnp.concatenate([in2, jnp.zeros((pad, D_EDGE), jnp.float32)])
  in2t = jnp.concatenate([in2p, in2p, in2p, in2p], axis=1)
  in2tr = jnp.reshape(in2t, (E_PAD * 16 // 128, 128))
  zrows = jnp.zeros((STRIPE, D_FEAT), jnp.float32)

  mesh = plsc.VectorSubcoreMesh(core_axis_name="c", subcore_axis_name="s",
                                num_cores=NUM_CORES,
                                num_subcores=NUM_SUBCORES)
  out4, _, _ = pl.kernel(
      _sc_body,
      out_type=(
          jax.ShapeDtypeStruct((4, N_NODES, D_FEAT), jnp.float32),
          jax.ShapeDtypeStruct((SROWS, 128), jnp.float32),
          jax.ShapeDtypeStruct((SROWS, 128), jnp.float32),
      ),
      mesh=mesh,
      scratch_types=[
          pltpu.VMEM_SHARED((N_NODES, D_FEAT), jnp.float32),
          pltpu.VMEM((4, EB), jnp.int32),
          pltpu.VMEM((4, EB), jnp.int32),
          pltpu.VMEM((2, 2 * WROWS, 128), jnp.float32),
          pltpu.VMEM((2, IROWS, 128), jnp.float32),
          pltpu.VMEM((2, EB, D_FEAT), jnp.float32),
          pltpu.VMEM((2, EB, D_FEAT), jnp.float32),
          pltpu.VMEM((WROWS, 128), jnp.float32),
          pltpu.SemaphoreType.DMA((2,)),
          pltpu.SemaphoreType.DMA((2,)),
          pltpu.SemaphoreType.DMA((2,)),
          pltpu.SemaphoreType.DMA,
          pltpu.SemaphoreType.DMA((2,)),
      ],
  )(in1, wqr[0], wqr[1], wqr[2], wqr[3], in2tr, sd, zrows)
  return _tc_interleave(out4)


_BN = 400


def _tc_interleave_body(x_ref, o_ref):
  for q in range(4):
    blk = jnp.reshape(x_ref[q], (_BN, 4, 32))
    blk = jnp.transpose(blk, (0, 2, 1))
    o_ref[:, 128 * q:128 * (q + 1)] = jnp.reshape(blk, (_BN, 128))


def _tc_interleave(x):
  return pl.pallas_call(
      _tc_interleave_body,
      grid=(N_NODES // _BN,),
      in_specs=[pl.BlockSpec((4, _BN, D_FEAT), lambda i: (0, i, 0))],
      out_specs=pl.BlockSpec((_BN, 4 * D_FEAT), lambda i: (i, 0)),
      out_shape=jax.ShapeDtypeStruct((N_NODES, 4 * D_FEAT), jnp.float32),
  )(x)


def kernel(in1, in2, weight, per_edge_src, per_edge_dst):
  return _fused_uvu(in1, in2, weight,
                    per_edge_src.astype(jnp.int32),
                    per_edge_dst.astype(jnp.int32))


# speedup vs baseline: 1.2398x; 1.2398x over previous
import jax
import jax.numpy as jnp
from jax import lax
from jax.experimental import pallas as pl
from jax.experimental.pallas import tpu as pltpu
from jax.experimental.pallas import tpu_sc as plsc

N_NODES = 10000
N_EDGES = 160000
D_FEAT = 128
D_EDGE = 4

NUM_CORES = 2
NUM_SUBCORES = 16
EB = 64
EDGES_PER_TILE = 10240
E_PAD = NUM_SUBCORES * EDGES_PER_TILE
NB = EDGES_PER_TILE // EB
WROWS = EB * 32 // 128
IROWS = EB * 16 // 128
SROWS = E_PAD * 32 // 128
STRIPE = 632
STRIPE_LAST = N_NODES - 15 * STRIPE


def _sc_body(in1_hbm, wq0, wq1, wq2, wq3, in2t_hbm, sd_hbm,
             zrows_hbm, out_hbm, stash0_hbm, stash1_hbm, accum,
             sidx, didx, wt, i2, rows, msg, stash,
             in_sems, g_sems, s_sems, st_sem, si_sems):
  cid = lax.axis_index("c")
  sid = lax.axis_index("s")
  ebase0 = sid * EDGES_PER_TILE
  rbase = pl.multiple_of(sid * STRIPE, 8)

  def stripe_copy(src_fn, dst_fn):
    @pl.when(sid < NUM_SUBCORES - 1)
    def _():
      pltpu.sync_copy(src_fn(rbase, STRIPE), dst_fn(rbase, STRIPE))

    @pl.when(sid == NUM_SUBCORES - 1)
    def _():
      base = (NUM_SUBCORES - 1) * STRIPE
      pltpu.sync_copy(src_fn(base, STRIPE_LAST), dst_fn(base, STRIPE_LAST))

  def eoff(b):
    return pl.multiple_of(ebase0 + b * EB, 8)

  def ioff(b):
    return pl.multiple_of((ebase0 + b * EB) // 8, 8)

  def woff(b):
    return pl.multiple_of((ebase0 + b * EB) // 4, 8)

  def issue_scatter(half, dslot):
    pltpu.async_copy(msg.at[half], accum.at[didx.at[dslot]],
                     s_sems.at[half], add=True)

  def wait_scatter(half, dslot):
    pltpu.make_async_copy(msg.at[half], accum.at[didx.at[dslot]],
                          s_sems.at[half]).wait()

  def epilogue(plane):
    wait_scatter(0, 2)
    wait_scatter(1, 3)
    plsc.subcore_barrier()
    stripe_copy(lambda base, n: accum.at[pl.ds(base, n), :],
                lambda base, n: out_hbm.at[plane, pl.ds(base, n), :])
    plsc.subcore_barrier()

  def zero_accum():
    stripe_copy(lambda base, n: zrows_hbm.at[pl.ds(0, n), :],
                lambda base, n: accum.at[pl.ds(base, n), :])
    plsc.subcore_barrier()

  def run_pass1(wtblA, wtblB, c0, stash_hbm, plane):
    zero_accum()

    def stage1_descs(b, half, dslot):
      sem = in_sems.at[half]
      return [
          pltpu.make_async_copy(sd_hbm.at[pl.ds(E_PAD + eoff(b), EB)],
                                didx.at[dslot], sem),
          pltpu.make_async_copy(in2t_hbm.at[pl.ds(ioff(b), IROWS), :],
                                i2.at[half], sem),
          pltpu.make_async_copy(wtblA.at[pl.ds(woff(b), WROWS), :],
                                wt.at[half, pl.ds(0, WROWS), :], sem),
          pltpu.make_async_copy(wtblB.at[pl.ds(woff(b), WROWS), :],
                                wt.at[half, pl.ds(WROWS, WROWS), :], sem),
      ]

    def stage1(b, half, dslot):
      for d in stage1_descs(b, half, dslot):
        d.start()

    def wait_stage1(b, half, dslot):
      for d in stage1_descs(b, half, dslot):
        d.wait()

    def sidx_desc(b, sslot):
      return pltpu.make_async_copy(sd_hbm.at[pl.ds(eoff(b), EB)],
                                   sidx.at[sslot], si_sems.at[sslot % 2])

    def gather_descs(sslot, half):
      return [
          pltpu.make_async_copy(in1_hbm.at[sidx.at[sslot, pl.ds(0, 32)]],
                                rows.at[half, pl.ds(0, 32), :],
                                g_sems.at[half]),
          pltpu.make_async_copy(in1_hbm.at[sidx.at[sslot, pl.ds(32, 32)]],
                                rows.at[half, pl.ds(32, 32), :],
                                g_sems.at[half]),
      ]

    def issue_gather(sslot, half):
      for d in gather_descs(sslot, half):
        d.start()

    def wait_gather(sslot, half):
      for d in gather_descs(sslot, half):
        d.wait()

    def stash_desc(b):
      return pltpu.make_async_copy(
          stash, stash_hbm.at[pl.ds(woff(b), WROWS), :], st_sem)

    def compute(half):
      @plsc.parallel_loop(0, EB // 8, 1, unroll=2)
      def _(g):
        for t in range(8):
          e = 8 * g + t
          wr = 2 * g + t // 4
          ca = 32 * (t % 4)
          wA0 = rows[half, e, c0:c0 + 16] * wt[half, wr, ca:ca + 16]
          wA1 = (rows[half, e, c0 + 16:c0 + 32]
                 * wt[half, wr, ca + 16:ca + 32])
          wB0 = (rows[half, e, c0 + 32:c0 + 48]
                 * wt[half, WROWS + wr, ca:ca + 16])
          wB1 = (rows[half, e, c0 + 48:c0 + 64]
                 * wt[half, WROWS + wr, ca + 16:ca + 32])
          stash[wr, ca:ca + 16] = wB0
          stash[wr, ca + 16:ca + 32] = wB1
          tv = i2[half, g, 16 * t:16 * t + 16]
          for j in range(4):
            s = tv[j]
            msg[half, e, 32 * j:32 * j + 16] = wA0 * s
            msg[half, e, 32 * j + 16:32 * j + 32] = wA1 * s

    sidx_desc(0, 0).start()
    sidx_desc(1, 1).start()
    stage1(0, 0, 0)
    wait_stage1(0, 0, 0)
    sidx_desc(0, 0).wait()
    issue_gather(0, 0)
    sidx_desc(1, 1).wait()
    issue_gather(1, 1)

    def body(i, carry):
      for h in range(4):
        b = 4 * i + h
        half = h % 2

        if h < 2:
          @pl.when(i > 0)
          def _():
            wait_scatter(half, (h + 2) % 4)
        else:
          wait_scatter(half, (h + 2) % 4)

        wait_gather(h, half)

        if h == 3:
          @pl.when(i < NB // 4 - 1)
          def _():
            stage1(b + 1, 1 - half, (h + 1) % 4)
        else:
          stage1(b + 1, 1 - half, (h + 1) % 4)

        if h >= 2:
          @pl.when(i < NB // 4 - 1)
          def _():
            sidx_desc(b + 2, (h + 2) % 4).start()
        else:
          sidx_desc(b + 2, (h + 2) % 4).start()

        if h == 0:
          @pl.when(i > 0)
          def _():
            stash_desc(b - 1).wait()
        else:
          stash_desc(b - 1).wait()

        compute(half)
        stash_desc(b).start()
        issue_scatter(half, h)

        if h == 3:
          @pl.when(i < NB // 4 - 1)
          def _():
            wait_stage1(b + 1, 1 - half, (h + 1) % 4)
        else:
          wait_stage1(b + 1, 1 - half, (h + 1) % 4)

        if h >= 2:
          @pl.when(i < NB // 4 - 1)
          def _():
            sidx_desc(b + 2, (h + 2) % 4).wait()
            issue_gather((h + 2) % 4, half)
        else:
          sidx_desc(b + 2, (h + 2) % 4).wait()
          issue_gather((h + 2) % 4, half)
      return carry

    lax.fori_loop(0, NB // 4, body, 0)
    stash_desc(NB - 1).wait()
    epilogue(plane)

  def run_pass2(stash_hbm, plane):
    zero_accum()

    def stage1_descs(b, half, dslot):
      sem = in_sems.at[half]
      return [
          pltpu.make_async_copy(sd_hbm.at[pl.ds(E_PAD + eoff(b), EB)],
                                didx.at[dslot], sem),
          pltpu.make_async_copy(in2t_hbm.at[pl.ds(ioff(b), IROWS), :],
                                i2.at[half], sem),
          pltpu.make_async_copy(stash_hbm.at[pl.ds(woff(b), WROWS), :],
                                wt.at[half, pl.ds(0, WROWS), :], sem),
      ]

    def stage1(b, half, dslot):
      for d in stage1_descs(b, half, dslot):
        d.start()

    def wait_stage1(b, half, dslot):
      for d in stage1_descs(b, half, dslot):
        d.wait()

    def compute(half):
      @plsc.parallel_loop(0, EB // 8, 1, unroll=2)
      def _(g):
        for t in range(8):
          e = 8 * g + t
          wr = 2 * g + t // 4
          ca = 32 * (t % 4)
          w0 = wt[half, wr, ca:ca + 16]
          w1 = wt[half, wr, ca + 16:ca + 32]
          tv = i2[half, g, 16 * t:16 * t + 16]
          for j in range(4):
            s = tv[j]
            msg[half, e, 32 * j:32 * j + 16] = w0 * s
            msg[half, e, 32 * j + 16:32 * j + 32] = w1 * s

    stage1(0, 0, 0)
    wait_stage1(0, 0, 0)

    def body(i, carry):
      for h in range(4):
        b = 4 * i + h
        half = h % 2

        if h < 2:
          @pl.when(i > 0)
          def _():
            wait_scatter(half, (h + 2) % 4)
        else:
          wait_scatter(half, (h + 2) % 4)

        if h == 3:
          @pl.when(i < NB // 4 - 1)
          def _():
            stage1(b + 1, 1 - half, (h + 1) % 4)
        else:
          stage1(b + 1, 1 - half, (h + 1) % 4)

        compute(half)
        issue_scatter(half, h)

        if h == 3:
          @pl.when(i < NB // 4 - 1)
          def _():
            wait_stage1(b + 1, 1 - half, (h + 1) % 4)
        else:
          wait_stage1(b + 1, 1 - half, (h + 1) % 4)
      return carry

    lax.fori_loop(0, NB // 4, body, 0)
    epilogue(plane)

  @pl.when(cid == 0)
  def _():
    run_pass1(wq0, wq1, 0, stash0_hbm, 0)
    run_pass2(stash0_hbm, 1)

  @pl.when(cid == 1)
  def _():
    run_pass1(wq2, wq3, 64, stash1_hbm, 2)
    run_pass2(stash1_hbm, 3)


@jax.jit
def _fused_uvu(in1, in2, weight, src, dst):
  pad = E_PAD - N_EDGES
  zpad = jnp.zeros((pad,), jnp.int32)
  sd = jnp.concatenate([src, zpad, dst, zpad])
  wp = jnp.concatenate([weight, jnp.zeros((pad, D_FEAT), jnp.float32)])
  wq = jnp.transpose(jnp.reshape(wp, (E_PAD, 4, 32)), (1, 0, 2))
  wqr = jnp.reshape(wq, (4, SROWS, 128))
  in2p = jnp.concatenate([in2, jnp.zeros((pad, D_EDGE), jnp.float32)])
  in2t = jnp.concatenate([in2p, in2p, in2p, in2p], axis=1)
  in2tr = jnp.reshape(in2t, (E_PAD * 16 // 128, 128))
  zrows = jnp.zeros((STRIPE, D_FEAT), jnp.float32)

  mesh = plsc.VectorSubcoreMesh(core_axis_name="c", subcore_axis_name="s",
                                num_cores=NUM_CORES,
                                num_subcores=NUM_SUBCORES)
  out4, _, _ = pl.kernel(
      _sc_body,
      out_type=(
          jax.ShapeDtypeStruct((4, N_NODES, D_FEAT), jnp.float32),
          jax.ShapeDtypeStruct((SROWS, 128), jnp.float32),
          jax.ShapeDtypeStruct((SROWS, 128), jnp.float32),
      ),
      mesh=mesh,
      scratch_types=[
          pltpu.VMEM_SHARED((N_NODES, D_FEAT), jnp.float32),
          pltpu.VMEM((4, EB), jnp.int32),
          pltpu.VMEM((4, EB), jnp.int32),
          pltpu.VMEM((2, 2 * WROWS, 128), jnp.float32),
          pltpu.VMEM((2, IROWS, 128), jnp.float32),
          pltpu.VMEM((2, EB, D_FEAT), jnp.float32),
          pltpu.VMEM((2, EB, D_FEAT), jnp.float32),
          pltpu.VMEM((WROWS, 128), jnp.float32),
          pltpu.SemaphoreType.DMA((2,)),
          pltpu.SemaphoreType.DMA((2,)),
          pltpu.SemaphoreType.DMA((2,)),
          pltpu.SemaphoreType.DMA,
          pltpu.SemaphoreType.DMA((2,)),
      ],
  )(in1, wqr[0], wqr[1], wqr[2], wqr[3], in2tr, sd, zrows)

  out = jnp.reshape(out4, (4, N_NODES, 4, 32))
  out = jnp.transpose(out, (1, 0, 3, 2))
  return jnp.reshape(out, (N_NODES, 4 * D_FEAT))


def kernel(in1, in2, weight, per_edge_src, per_edge_dst):
  return _fused_uvu(in1, in2, weight,
                    per_edge_src.astype(jnp.int32),
                    per_edge_dst.astype(jnp.int32))
